# Initial kernel scaffold; baseline (speedup 1.0000x reference)
import jax
import jax.numpy as jnp
from jax.experimental import pallas as pl


def kernel(points, indices):
    raise NotImplementedError("write your pallas kernel here")


# speedup vs baseline: 1.6644x; 1.6644x over previous
import functools

import jax
import jax.numpy as jnp
from jax import lax
from jax.experimental import pallas as pl
from jax.experimental.pallas import tpu as pltpu
from jax.experimental.pallas import tpu_sc as plsc

N, C, P, M = 32, 128, 16384, 4096
L = 16
NC, NS = 2, 16
NW = NC * NS


def _gather_body(points_hbm, idx_hbm, out_hbm, idx_v, row_v, out_v):
    n = lax.axis_index("s") * NC + lax.axis_index("c")
    pltpu.sync_copy(idx_hbm.at[n], idx_v)

    def c_loop(c, carry):
        pltpu.sync_copy(points_hbm.at[n, c], row_v)

        def g_loop(i, carry2):
            base = pl.multiple_of(i * L, L)
            idx = idx_v[pl.ds(base, L)]
            out_v[pl.ds(base, L)] = plsc.load_gather(row_v, [idx])
            return carry2

        lax.fori_loop(0, M // L, g_loop, 0, unroll=4)
        pltpu.sync_copy(out_v, out_hbm.at[n, c])
        return carry

    lax.fori_loop(0, C, c_loop, 0)


@jax.jit
def kernel(points, indices):
    idx32 = indices.astype(jnp.int32)
    mesh = plsc.VectorSubcoreMesh(core_axis_name="c", subcore_axis_name="s")
    run = functools.partial(
        pl.kernel,
        mesh=mesh,
        out_type=jax.ShapeDtypeStruct((N, C, M), jnp.float32),
        scratch_types=[
            pltpu.VMEM((M,), jnp.int32),
            pltpu.VMEM((P,), jnp.float32),
            pltpu.VMEM((M,), jnp.float32),
        ],
        compiler_params=pltpu.CompilerParams(needs_layout_passes=False),
    )(_gather_body)
    return run(points, idx32)


# speedup vs baseline: 2.7446x; 1.6490x over previous
import functools

import jax
import jax.numpy as jnp
from jax import lax
from jax.experimental import pallas as pl
from jax.experimental.pallas import tpu as pltpu
from jax.experimental.pallas import tpu_sc as plsc

N, C, P, M = 32, 128, 16384, 4096
L = 16
NC, NS = 2, 16
NW = NC * NS


def _gather_body(points_hbm, idx_hbm, out_hbm, idx_v, row_v0, row_v1,
                 out_v0, out_v1, in_sem0, in_sem1, out_sem0, out_sem1):
    n = lax.axis_index("s") * NC + lax.axis_index("c")
    rows = (row_v0, row_v1)
    outs = (out_v0, out_v1)
    in_sems = (in_sem0, in_sem1)
    out_sems = (out_sem0, out_sem1)

    pltpu.sync_copy(idx_hbm.at[n], idx_v)

    for b in range(2):
        pltpu.async_copy(points_hbm.at[n, b], rows[b], in_sems[b])

    def outer(c0, carry):
        for b in range(2):
            c = c0 * 2 + b
            pltpu.make_async_copy(
                points_hbm.at[n, c], rows[b], in_sems[b]).wait()

            @pl.when(c0 > 0)
            def _wait_out():
                pltpu.make_async_copy(
                    outs[b], out_hbm.at[n, c - 2], out_sems[b]).wait()

            def g_loop(i, carry2):
                base = pl.multiple_of(i * L, L)
                idx = idx_v[pl.ds(base, L)]
                outs[b][pl.ds(base, L)] = plsc.load_gather(rows[b], [idx])
                return carry2

            lax.fori_loop(0, M // L, g_loop, 0, unroll=8)

            pltpu.async_copy(outs[b], out_hbm.at[n, c], out_sems[b])

            @pl.when(c + 2 < C)
            def _next_in():
                pltpu.async_copy(
                    points_hbm.at[n, c + 2], rows[b], in_sems[b])
        return carry

    lax.fori_loop(0, C // 2, outer, 0)

    for b in range(2):
        pltpu.make_async_copy(
            outs[b], out_hbm.at[n, C - 2 + b], out_sems[b]).wait()


@jax.jit
def kernel(points, indices):
    idx32 = indices.astype(jnp.int32)
    mesh = plsc.VectorSubcoreMesh(core_axis_name="c", subcore_axis_name="s")
    run = functools.partial(
        pl.kernel,
        mesh=mesh,
        out_type=jax.ShapeDtypeStruct((N, C, M), jnp.float32),
        scratch_types=[
            pltpu.VMEM((M,), jnp.int32),
            pltpu.VMEM((P,), jnp.float32),
            pltpu.VMEM((P,), jnp.float32),
            pltpu.VMEM((M,), jnp.float32),
            pltpu.VMEM((M,), jnp.float32),
            pltpu.SemaphoreType.DMA,
            pltpu.SemaphoreType.DMA,
            pltpu.SemaphoreType.DMA,
            pltpu.SemaphoreType.DMA,
        ],
        compiler_params=pltpu.CompilerParams(needs_layout_passes=False),
    )(_gather_body)
    return run(points, idx32)


# speedup vs baseline: 5.5026x; 2.0049x over previous
import functools

import jax
import jax.numpy as jnp
from jax import lax
from jax.experimental import pallas as pl
from jax.experimental.pallas import tpu as pltpu
from jax.experimental.pallas import tpu_sc as plsc

N, C, P, M = 32, 128, 16384, 4096
L = 16
NC, NS = 2, 16
NW = NC * NS


def _gather_body(points_hbm, idx_hbm, out_hbm, idx_v, row_v0, row_v1,
                 out_v0, out_v1, in_sem0, in_sem1, out_sem0, out_sem1):
    n = lax.axis_index("s") * NC + lax.axis_index("c")
    rows = (row_v0, row_v1)
    outs = (out_v0, out_v1)
    in_sems = (in_sem0, in_sem1)
    out_sems = (out_sem0, out_sem1)

    pltpu.sync_copy(idx_hbm.at[n], idx_v)

    for b in range(2):
        pltpu.async_copy(points_hbm.at[n, b], rows[b], in_sems[b])

    def outer(c0, carry):
        for b in range(2):
            c = c0 * 2 + b
            pltpu.make_async_copy(
                points_hbm.at[n, c], rows[b], in_sems[b]).wait()

            @pl.when(c0 > 0)
            def _wait_out():
                pltpu.make_async_copy(
                    outs[b], out_hbm.at[n, c - 2], out_sems[b]).wait()

            @plsc.parallel_loop(0, M, L, unroll=8)
            def g_loop(i):
                base = pl.multiple_of(i, L)
                idx = idx_v[pl.ds(base, L)]
                outs[b][pl.ds(base, L)] = plsc.load_gather(rows[b], [idx])

            pltpu.async_copy(outs[b], out_hbm.at[n, c], out_sems[b])

            @pl.when(c + 2 < C)
            def _next_in():
                pltpu.async_copy(
                    points_hbm.at[n, c + 2], rows[b], in_sems[b])
        return carry

    lax.fori_loop(0, C // 2, outer, 0)

    for b in range(2):
        pltpu.make_async_copy(
            outs[b], out_hbm.at[n, C - 2 + b], out_sems[b]).wait()


@jax.jit
def kernel(points, indices):
    idx32 = indices.astype(jnp.int32)
    mesh = plsc.VectorSubcoreMesh(core_axis_name="c", subcore_axis_name="s")
    run = functools.partial(
        pl.kernel,
        mesh=mesh,
        out_type=jax.ShapeDtypeStruct((N, C, M), jnp.float32),
        scratch_types=[
            pltpu.VMEM((M,), jnp.int32),
            pltpu.VMEM((P,), jnp.float32),
            pltpu.VMEM((P,), jnp.float32),
            pltpu.VMEM((M,), jnp.float32),
            pltpu.VMEM((M,), jnp.float32),
            pltpu.SemaphoreType.DMA,
            pltpu.SemaphoreType.DMA,
            pltpu.SemaphoreType.DMA,
            pltpu.SemaphoreType.DMA,
        ],
        compiler_params=pltpu.CompilerParams(needs_layout_passes=False),
    )(_gather_body)
    return run(points, idx32)
